# baseline (speedup 1.0000x reference)
import functools

import jax
import jax.numpy as jnp
from jax import lax
from jax.experimental import pallas as pl
from jax.experimental.pallas import tpu as pltpu
from jax.experimental.pallas import tpu_sc as plsc

N = 10000
DEG = 16
D = 256
L = 3
H4 = 4 * D
NB = 200
GW = 128
NP = 10240
_PREC = lax.Precision.DEFAULT


def _sigmoid(x):
    return 0.5 * jnp.tanh(0.5 * x) + 0.5


def _sc_gather(table, idx3):
    nwin = idx3.shape[0]
    w = table.shape[1]
    b = nwin * GW
    mesh = plsc.VectorSubcoreMesh(core_axis_name="c", subcore_axis_name="s")

    @functools.partial(
        pl.kernel,
        out_type=jax.ShapeDtypeStruct((b, w), table.dtype),
        mesh=mesh,
    )
    def gk(tab_hbm, idx_hbm, out_hbm):
        def body(i_vmem, o_vmem):
            pltpu.sync_copy(tab_hbm.at[i_vmem.at[0, 0]], o_vmem)

        pltpu.emit_pipeline(
            body,
            grid=(nwin,),
            in_specs=[pl.BlockSpec((1, 1, GW), lambda i: (i, 0, 0))],
            out_specs=[pl.BlockSpec((GW, w), lambda i: (i, 0))],
            core_axis_name=("c", "s"),
            dimension_semantics=(pltpu.PARALLEL,),
        )(idx_hbm, out_hbm)

    return gk(table, idx3)


def _layer_body(h_ref, g_ref, wih_ref, whh_ref, wself_ref, wneigh_ref,
                bias_ref, scale_ref, shift_ref, out_ref, *maybe_stats,
                apply_norm, want_stats, want_ls):
    h_blk = h_ref[...]
    g_blk = g_ref[...].reshape(DEG * NB, D)
    if apply_norm:
        sc = scale_ref[0]
        sh = shift_ref[0]
        h_blk = jnp.maximum(h_blk * sc + sh, 0.0)
        g_blk = jnp.maximum(g_blk * sc + sh, 0.0)
    h_blk = h_blk.astype(jnp.bfloat16)
    g_blk = g_blk.astype(jnp.bfloat16)
    pin = jnp.dot(g_blk, wih_ref[...],
                  preferred_element_type=jnp.float32) + bias_ref[0]
    pin = pin.reshape(DEG, NB, H4)
    whh = whh_ref[...]

    z = jnp.zeros((NB, D), jnp.float32)
    hs, c = z, z
    for t in range(DEG):
        gt = pin[t] + jnp.dot(hs, whh, precision=_PREC)
        i_ = _sigmoid(gt[:, 0:D])
        f_ = _sigmoid(gt[:, D:2 * D])
        g_ = jnp.tanh(gt[:, 2 * D:3 * D])
        o_ = _sigmoid(gt[:, 3 * D:4 * D])
        c = f_ * c + i_ * g_
        hs = o_ * jnp.tanh(c)

    out = (jnp.dot(h_blk, wself_ref[...], preferred_element_type=jnp.float32)
           + jnp.dot(hs.astype(jnp.bfloat16), wneigh_ref[...],
                     preferred_element_type=jnp.float32))
    if want_ls:
        m = jnp.max(out, axis=-1, keepdims=True)
        e = jnp.exp(out - m)
        out = out - m - jnp.log(jnp.sum(e, axis=-1, keepdims=True))
        out_ref[...] = out
    else:
        out_ref[...] = out
    if want_stats:
        stats_ref = maybe_stats[0]
        s1 = jnp.sum(out, axis=0, keepdims=True)
        s2 = jnp.sum(out * out, axis=0, keepdims=True)
        stats_ref[...] = jnp.concatenate([s1, s2], axis=0)[None]


def _tc_layer(h, g3, wih_t, whh_t, wself_t, wneigh_t, bias, scale, shift,
              *, cn, cnp, koff, apply_norm, want_stats, want_ls):
    nblk = cn // NB
    body = functools.partial(_layer_body, apply_norm=apply_norm,
                             want_stats=want_stats, want_ls=want_ls)
    in_specs = [
        pl.BlockSpec((NB, D), lambda i: (i + koff, 0)),
        pl.BlockSpec((DEG, NB, D), lambda i: (0, i, 0)),
        pl.BlockSpec((D, H4), lambda i: (0, 0)),
        pl.BlockSpec((D, H4), lambda i: (0, 0)),
        pl.BlockSpec((D, D), lambda i: (0, 0)),
        pl.BlockSpec((D, D), lambda i: (0, 0)),
        pl.BlockSpec((1, H4), lambda i: (0, 0)),
        pl.BlockSpec((1, D), lambda i: (0, 0)),
        pl.BlockSpec((1, D), lambda i: (0, 0)),
    ]
    if want_stats:
        out_shape = [jax.ShapeDtypeStruct((cn, D), jnp.float32),
                     jax.ShapeDtypeStruct((nblk, 2, D), jnp.float32)]
        out_specs = [pl.BlockSpec((NB, D), lambda i: (i, 0)),
                     pl.BlockSpec((1, 2, D), lambda i: (i, 0, 0))]
    else:
        out_shape = jax.ShapeDtypeStruct((cn, D), jnp.float32)
        out_specs = pl.BlockSpec((NB, D), lambda i: (i, 0))
    return pl.pallas_call(
        body,
        grid=(nblk,),
        in_specs=in_specs,
        out_specs=out_specs,
        out_shape=out_shape,
    )(h, g3, wih_t, whh_t, wself_t, wneigh_t, bias, scale, shift)


NC = 10


def kernel(x, edge_index, Wself, Wneigh, Wih, Whh, bih, bhh, gamma, beta):
    src = edge_index[0]
    cn = N // NC
    cnp = -(-cn // 256) * 256
    src_mat = src.reshape(N, DEG).T
    src_chunks = []
    for k in range(NC):
        part = src_mat[:, k * cn:(k + 1) * cn]
        part = jnp.pad(part, ((0, 0), (0, cnp - cn)))
        src_chunks.append(part.reshape(DEG * cnp // GW, 1, GW))
    wih_t = jnp.swapaxes(Wih, 1, 2).astype(jnp.bfloat16)
    whh_t = jnp.swapaxes(Whh, 1, 2)
    wself_t = jnp.swapaxes(Wself, 1, 2).astype(jnp.bfloat16)
    wneigh_t = jnp.swapaxes(Wneigh, 1, 2).astype(jnp.bfloat16)
    bias = (bih + bhh).reshape(L, 1, H4)

    h = x
    scale = jnp.ones((1, D), jnp.float32)
    shift = jnp.zeros((1, D), jnp.float32)
    for l in range(L):
        last = l == L - 1
        outs, stats_parts = [], []
        for k in range(NC):
            g3 = _sc_gather(h, src_chunks[k]).reshape(DEG, cnp, D)
            res = _tc_layer(h, g3, wih_t[l], whh_t[l], wself_t[l],
                            wneigh_t[l], bias[l], scale, shift,
                            cn=cn, cnp=cnp, koff=k * (cn // NB),
                            apply_norm=(l > 0), want_stats=not last,
                            want_ls=last)
            if last:
                outs.append(res)
            else:
                outs.append(res[0])
                stats_parts.append(res[1])
        h = jnp.concatenate(outs, axis=0)
        if last:
            return h
        s = jnp.sum(jnp.concatenate(stats_parts, axis=0), axis=0)
        mu = s[0] / N
        var = s[1] / N - mu * mu
        inv = lax.rsqrt(var + 1e-5)
        scale = (gamma[l] * inv).reshape(1, D)
        shift = (beta[l] - mu * gamma[l] * inv).reshape(1, D)
    return None


# speedup vs baseline: 1.0335x; 1.0335x over previous
import functools

import jax
import jax.numpy as jnp
from jax import lax
from jax.experimental import pallas as pl
from jax.experimental.pallas import tpu as pltpu
from jax.experimental.pallas import tpu_sc as plsc

N = 10000
DEG = 16
D = 256
L = 3
H4 = 4 * D
NB = 200
GW = 128
NP = 10240
_PREC = lax.Precision.DEFAULT


def _sigmoid(x):
    return 0.5 * jnp.tanh(0.5 * x) + 0.5


def _sc_gather(table, idx3):
    nwin = idx3.shape[0]
    w = table.shape[1]
    b = nwin * GW
    mesh = plsc.VectorSubcoreMesh(core_axis_name="c", subcore_axis_name="s")

    @functools.partial(
        pl.kernel,
        out_type=jax.ShapeDtypeStruct((b, w), table.dtype),
        mesh=mesh,
    )
    def gk(tab_hbm, idx_hbm, out_hbm):
        def body(i_vmem, o_vmem):
            pltpu.sync_copy(tab_hbm.at[i_vmem.at[0, 0]], o_vmem)

        pltpu.emit_pipeline(
            body,
            grid=(nwin,),
            in_specs=[pl.BlockSpec((1, 1, GW), lambda i: (i, 0, 0))],
            out_specs=[pl.BlockSpec((GW, w), lambda i: (i, 0))],
            core_axis_name=("c", "s"),
            dimension_semantics=(pltpu.PARALLEL,),
        )(idx_hbm, out_hbm)

    return gk(table, idx3)


def _layer_body(h_ref, g_ref, wih_ref, whh_ref, wself_ref, wneigh_ref,
                bias_ref, scale_ref, shift_ref, out_ref, *maybe_stats,
                apply_norm, want_stats, want_ls):
    h_blk = h_ref[...]
    g_blk = g_ref[...].reshape(DEG * NB, D)
    if apply_norm:
        sc = scale_ref[0]
        sh = shift_ref[0]
        h_blk = jnp.maximum(h_blk * sc + sh, 0.0)
        g_blk = jnp.maximum(g_blk * sc + sh, 0.0)
    h_blk = h_blk.astype(jnp.bfloat16)
    g_blk = g_blk.astype(jnp.bfloat16)
    pin = jnp.dot(g_blk, wih_ref[...],
                  preferred_element_type=jnp.float32) + bias_ref[0]
    pin = pin.reshape(DEG, NB, H4)
    whh = whh_ref[...]

    z = jnp.zeros((NB, D), jnp.float32)
    hs, c = z, z
    for t in range(DEG):
        gt = pin[t] + jnp.dot(hs, whh, precision=_PREC)
        i_ = _sigmoid(gt[:, 0:D])
        f_ = _sigmoid(gt[:, D:2 * D])
        g_ = jnp.tanh(gt[:, 2 * D:3 * D])
        o_ = _sigmoid(gt[:, 3 * D:4 * D])
        c = f_ * c + i_ * g_
        hs = o_ * jnp.tanh(c)

    out = (jnp.dot(h_blk, wself_ref[...], preferred_element_type=jnp.float32)
           + jnp.dot(hs.astype(jnp.bfloat16), wneigh_ref[...],
                     preferred_element_type=jnp.float32))
    if want_ls:
        m = jnp.max(out, axis=-1, keepdims=True)
        e = jnp.exp(out - m)
        out = out - m - jnp.log(jnp.sum(e, axis=-1, keepdims=True))
        out_ref[...] = out
    else:
        out_ref[...] = out
    if want_stats:
        stats_ref = maybe_stats[0]
        s1 = jnp.sum(out, axis=0, keepdims=True)
        s2 = jnp.sum(out * out, axis=0, keepdims=True)
        stats_ref[...] = jnp.concatenate([s1, s2], axis=0)[None]


def _tc_layer(h, g3, wih_t, whh_t, wself_t, wneigh_t, bias, scale, shift,
              *, cn, cnp, koff, apply_norm, want_stats, want_ls):
    nblk = cn // NB
    body = functools.partial(_layer_body, apply_norm=apply_norm,
                             want_stats=want_stats, want_ls=want_ls)
    in_specs = [
        pl.BlockSpec((NB, D), lambda i: (i + koff, 0)),
        pl.BlockSpec((DEG, NB, D), lambda i: (0, i, 0)),
        pl.BlockSpec((D, H4), lambda i: (0, 0)),
        pl.BlockSpec((D, H4), lambda i: (0, 0)),
        pl.BlockSpec((D, D), lambda i: (0, 0)),
        pl.BlockSpec((D, D), lambda i: (0, 0)),
        pl.BlockSpec((1, H4), lambda i: (0, 0)),
        pl.BlockSpec((1, D), lambda i: (0, 0)),
        pl.BlockSpec((1, D), lambda i: (0, 0)),
    ]
    if want_stats:
        out_shape = [jax.ShapeDtypeStruct((cn, D), jnp.float32),
                     jax.ShapeDtypeStruct((nblk, 2, D), jnp.float32)]
        out_specs = [pl.BlockSpec((NB, D), lambda i: (i, 0)),
                     pl.BlockSpec((1, 2, D), lambda i: (i, 0, 0))]
    else:
        out_shape = jax.ShapeDtypeStruct((cn, D), jnp.float32)
        out_specs = pl.BlockSpec((NB, D), lambda i: (i, 0))
    return pl.pallas_call(
        body,
        grid=(nblk,),
        in_specs=in_specs,
        out_specs=out_specs,
        out_shape=out_shape,
    )(h, g3, wih_t, whh_t, wself_t, wneigh_t, bias, scale, shift)


NC = 5


def kernel(x, edge_index, Wself, Wneigh, Wih, Whh, bih, bhh, gamma, beta):
    src = edge_index[0]
    cn = N // NC
    cnp = -(-cn // 256) * 256
    src_mat = src.reshape(N, DEG).T
    src_chunks = []
    for k in range(NC):
        part = src_mat[:, k * cn:(k + 1) * cn]
        part = jnp.pad(part, ((0, 0), (0, cnp - cn)))
        src_chunks.append(part.reshape(DEG * cnp // GW, 1, GW))
    wih_t = jnp.swapaxes(Wih, 1, 2).astype(jnp.bfloat16)
    whh_t = jnp.swapaxes(Whh, 1, 2)
    wself_t = jnp.swapaxes(Wself, 1, 2).astype(jnp.bfloat16)
    wneigh_t = jnp.swapaxes(Wneigh, 1, 2).astype(jnp.bfloat16)
    bias = (bih + bhh).reshape(L, 1, H4)

    h = x
    scale = jnp.ones((1, D), jnp.float32)
    shift = jnp.zeros((1, D), jnp.float32)
    for l in range(L):
        last = l == L - 1
        outs, stats_parts = [], []
        for k in range(NC):
            g3 = _sc_gather(h, src_chunks[k]).reshape(DEG, cnp, D)
            res = _tc_layer(h, g3, wih_t[l], whh_t[l], wself_t[l],
                            wneigh_t[l], bias[l], scale, shift,
                            cn=cn, cnp=cnp, koff=k * (cn // NB),
                            apply_norm=(l > 0), want_stats=not last,
                            want_ls=last)
            if last:
                outs.append(res)
            else:
                outs.append(res[0])
                stats_parts.append(res[1])
        h = jnp.concatenate(outs, axis=0)
        if last:
            return h
        s = jnp.sum(jnp.concatenate(stats_parts, axis=0), axis=0)
        mu = s[0] / N
        var = s[1] / N - mu * mu
        inv = lax.rsqrt(var + 1e-5)
        scale = (gamma[l] * inv).reshape(1, D)
        shift = (beta[l] - mu * gamma[l] * inv).reshape(1, D)
    return None


# speedup vs baseline: 1.0397x; 1.0060x over previous
import functools

import jax
import jax.numpy as jnp
from jax import lax
from jax.experimental import pallas as pl
from jax.experimental.pallas import tpu as pltpu
from jax.experimental.pallas import tpu_sc as plsc

N = 10000
DEG = 16
D = 256
L = 3
H4 = 4 * D
NB = 200
GW = 128
NP = 10240
_PREC = lax.Precision.DEFAULT


def _sigmoid(x):
    return 0.5 * jnp.tanh(0.5 * x) + 0.5


def _sc_gather(table, idx3):
    nwin = idx3.shape[0]
    w = table.shape[1]
    b = nwin * GW
    nw_workers = 32
    wpw = nwin // nw_workers
    assert nwin % nw_workers == 0
    mesh = plsc.VectorSubcoreMesh(core_axis_name="c", subcore_axis_name="s")

    @functools.partial(
        pl.kernel,
        out_type=jax.ShapeDtypeStruct((b, w), table.dtype),
        mesh=mesh,
        scratch_types=[
            pltpu.VMEM((wpw, 1, GW), jnp.int32),
            pltpu.VMEM((2, GW, w), table.dtype),
            pltpu.SemaphoreType.DMA,
            pltpu.SemaphoreType.DMA,
            pltpu.SemaphoreType.DMA,
            pltpu.SemaphoreType.DMA,
        ],
    )
    def gk(tab_hbm, idx_hbm, out_hbm, idx_v, rows_v, sg0, sg1, so0, so1):
        wid = lax.axis_index("s") * 2 + lax.axis_index("c")
        base = wid * wpw
        pltpu.sync_copy(idx_hbm.at[pl.ds(base, wpw)], idx_v)
        sg = [sg0, sg1]
        so = [so0, so1]
        gathers = [None, None]
        stores = [None, None]
        for g in range(min(2, wpw)):
            gathers[g] = pltpu.async_copy(
                tab_hbm.at[idx_v.at[g, 0]], rows_v.at[g], sg[g])
        for g in range(wpw):
            slot = g % 2
            gathers[slot].wait()
            stores[slot] = pltpu.async_copy(
                rows_v.at[slot], out_hbm.at[pl.ds((base + g) * GW, GW)],
                so[slot])
            if g + 2 < wpw:
                stores[slot].wait()
                gathers[slot] = pltpu.async_copy(
                    tab_hbm.at[idx_v.at[g + 2, 0]], rows_v.at[slot], sg[slot])
        for slot in range(min(2, wpw)):
            if stores[slot] is not None:
                stores[slot].wait()

    return gk(table, idx3)


def _layer_body(h_ref, g_ref, wih_ref, whh_ref, wself_ref, wneigh_ref,
                bias_ref, scale_ref, shift_ref, out_ref, *maybe_stats,
                apply_norm, want_stats, want_ls):
    h_blk = h_ref[...]
    g_blk = g_ref[...].reshape(DEG * NB, D)
    if apply_norm:
        sc = scale_ref[0]
        sh = shift_ref[0]
        h_blk = jnp.maximum(h_blk * sc + sh, 0.0)
        g_blk = jnp.maximum(g_blk * sc + sh, 0.0)
    h_blk = h_blk.astype(jnp.bfloat16)
    g_blk = g_blk.astype(jnp.bfloat16)
    pin = jnp.dot(g_blk, wih_ref[...],
                  preferred_element_type=jnp.float32) + bias_ref[0]
    pin = pin.reshape(DEG, NB, H4)
    whh = whh_ref[...]

    z = jnp.zeros((NB, D), jnp.float32)
    hs, c = z, z
    for t in range(DEG):
        gt = pin[t] + jnp.dot(hs, whh, precision=_PREC)
        i_ = _sigmoid(gt[:, 0:D])
        f_ = _sigmoid(gt[:, D:2 * D])
        g_ = jnp.tanh(gt[:, 2 * D:3 * D])
        o_ = _sigmoid(gt[:, 3 * D:4 * D])
        c = f_ * c + i_ * g_
        hs = o_ * jnp.tanh(c)

    out = (jnp.dot(h_blk, wself_ref[...], preferred_element_type=jnp.float32)
           + jnp.dot(hs.astype(jnp.bfloat16), wneigh_ref[...],
                     preferred_element_type=jnp.float32))
    if want_ls:
        m = jnp.max(out, axis=-1, keepdims=True)
        e = jnp.exp(out - m)
        out = out - m - jnp.log(jnp.sum(e, axis=-1, keepdims=True))
        out_ref[...] = out
    else:
        out_ref[...] = out
    if want_stats:
        stats_ref = maybe_stats[0]
        s1 = jnp.sum(out, axis=0, keepdims=True)
        s2 = jnp.sum(out * out, axis=0, keepdims=True)
        stats_ref[...] = jnp.concatenate([s1, s2], axis=0)[None]


def _tc_layer(h, g3, wih_t, whh_t, wself_t, wneigh_t, bias, scale, shift,
              *, cn, cnp, koff, apply_norm, want_stats, want_ls):
    nblk = cn // NB
    body = functools.partial(_layer_body, apply_norm=apply_norm,
                             want_stats=want_stats, want_ls=want_ls)
    in_specs = [
        pl.BlockSpec((NB, D), lambda i: (i + koff, 0)),
        pl.BlockSpec((DEG, NB, D), lambda i: (0, i, 0)),
        pl.BlockSpec((D, H4), lambda i: (0, 0)),
        pl.BlockSpec((D, H4), lambda i: (0, 0)),
        pl.BlockSpec((D, D), lambda i: (0, 0)),
        pl.BlockSpec((D, D), lambda i: (0, 0)),
        pl.BlockSpec((1, H4), lambda i: (0, 0)),
        pl.BlockSpec((1, D), lambda i: (0, 0)),
        pl.BlockSpec((1, D), lambda i: (0, 0)),
    ]
    if want_stats:
        out_shape = [jax.ShapeDtypeStruct((cn, D), jnp.float32),
                     jax.ShapeDtypeStruct((nblk, 2, D), jnp.float32)]
        out_specs = [pl.BlockSpec((NB, D), lambda i: (i, 0)),
                     pl.BlockSpec((1, 2, D), lambda i: (i, 0, 0))]
    else:
        out_shape = jax.ShapeDtypeStruct((cn, D), jnp.float32)
        out_specs = pl.BlockSpec((NB, D), lambda i: (i, 0))
    return pl.pallas_call(
        body,
        grid=(nblk,),
        in_specs=in_specs,
        out_specs=out_specs,
        out_shape=out_shape,
    )(h, g3, wih_t, whh_t, wself_t, wneigh_t, bias, scale, shift)


NC = 5


def kernel(x, edge_index, Wself, Wneigh, Wih, Whh, bih, bhh, gamma, beta):
    src = edge_index[0]
    cn = N // NC
    cnp = -(-cn // 256) * 256
    src_mat = src.reshape(N, DEG).T
    src_chunks = []
    for k in range(NC):
        part = src_mat[:, k * cn:(k + 1) * cn]
        part = jnp.pad(part, ((0, 0), (0, cnp - cn)))
        src_chunks.append(part.reshape(DEG * cnp // GW, 1, GW))
    wih_t = jnp.swapaxes(Wih, 1, 2).astype(jnp.bfloat16)
    whh_t = jnp.swapaxes(Whh, 1, 2)
    wself_t = jnp.swapaxes(Wself, 1, 2).astype(jnp.bfloat16)
    wneigh_t = jnp.swapaxes(Wneigh, 1, 2).astype(jnp.bfloat16)
    bias = (bih + bhh).reshape(L, 1, H4)

    h = x
    scale = jnp.ones((1, D), jnp.float32)
    shift = jnp.zeros((1, D), jnp.float32)
    for l in range(L):
        last = l == L - 1
        outs, stats_parts = [], []
        for k in range(NC):
            g3 = _sc_gather(h, src_chunks[k]).reshape(DEG, cnp, D)
            res = _tc_layer(h, g3, wih_t[l], whh_t[l], wself_t[l],
                            wneigh_t[l], bias[l], scale, shift,
                            cn=cn, cnp=cnp, koff=k * (cn // NB),
                            apply_norm=(l > 0), want_stats=not last,
                            want_ls=last)
            if last:
                outs.append(res)
            else:
                outs.append(res[0])
                stats_parts.append(res[1])
        h = jnp.concatenate(outs, axis=0)
        if last:
            return h
        s = jnp.sum(jnp.concatenate(stats_parts, axis=0), axis=0)
        mu = s[0] / N
        var = s[1] / N - mu * mu
        inv = lax.rsqrt(var + 1e-5)
        scale = (gamma[l] * inv).reshape(1, D)
        shift = (beta[l] - mu * gamma[l] * inv).reshape(1, D)
    return None


# speedup vs baseline: 1.0879x; 1.0464x over previous
import functools

import jax
import jax.numpy as jnp
from jax import lax
from jax.experimental import pallas as pl
from jax.experimental.pallas import tpu as pltpu
from jax.experimental.pallas import tpu_sc as plsc

N = 10000
DEG = 16
D = 256
L = 3
H4 = 4 * D
NB = 400
GW = 128
NP = 10240
_PREC = lax.Precision.DEFAULT


def _sigmoid(x):
    return 0.5 * jnp.tanh(0.5 * x) + 0.5


def _sc_gather(table, idx3):
    nwin = idx3.shape[0]
    w = table.shape[1]
    b = nwin * GW
    nw_workers = 32
    wpw = nwin // nw_workers
    assert nwin % nw_workers == 0
    mesh = plsc.VectorSubcoreMesh(core_axis_name="c", subcore_axis_name="s")

    @functools.partial(
        pl.kernel,
        out_type=jax.ShapeDtypeStruct((b, w), table.dtype),
        mesh=mesh,
        scratch_types=[
            pltpu.VMEM((wpw, 1, GW), jnp.int32),
            pltpu.VMEM((2, GW, w), table.dtype),
            pltpu.SemaphoreType.DMA,
            pltpu.SemaphoreType.DMA,
            pltpu.SemaphoreType.DMA,
            pltpu.SemaphoreType.DMA,
        ],
    )
    def gk(tab_hbm, idx_hbm, out_hbm, idx_v, rows_v, sg0, sg1, so0, so1):
        wid = lax.axis_index("s") * 2 + lax.axis_index("c")
        base = wid * wpw
        pltpu.sync_copy(idx_hbm.at[pl.ds(base, wpw)], idx_v)
        sg = [sg0, sg1]
        so = [so0, so1]
        gathers = [None, None]
        stores = [None, None]
        for g in range(min(2, wpw)):
            gathers[g] = pltpu.async_copy(
                tab_hbm.at[idx_v.at[g, 0]], rows_v.at[g], sg[g])
        for g in range(wpw):
            slot = g % 2
            gathers[slot].wait()
            stores[slot] = pltpu.async_copy(
                rows_v.at[slot], out_hbm.at[pl.ds((base + g) * GW, GW)],
                so[slot])
            if g + 2 < wpw:
                stores[slot].wait()
                gathers[slot] = pltpu.async_copy(
                    tab_hbm.at[idx_v.at[g + 2, 0]], rows_v.at[slot], sg[slot])
        for slot in range(min(2, wpw)):
            if stores[slot] is not None:
                stores[slot].wait()

    return gk(table, idx3)


def _layer_body(h_ref, g_ref, wih_ref, whh_ref, wself_ref, wneigh_ref,
                bias_ref, scale_ref, shift_ref, out_ref, *maybe_stats,
                apply_norm, want_stats, want_ls):
    h_blk = h_ref[...]
    g_blk = g_ref[...].reshape(DEG * NB, D)
    if apply_norm:
        sc = scale_ref[0]
        sh = shift_ref[0]
        h_blk = jnp.maximum(h_blk * sc + sh, 0.0)
        g_blk = jnp.maximum(g_blk * sc + sh, 0.0)
    h_blk = h_blk.astype(jnp.bfloat16)
    g_blk = g_blk.astype(jnp.bfloat16)
    pin = jnp.dot(g_blk, wih_ref[...],
                  preferred_element_type=jnp.float32) + bias_ref[0]
    pin = pin.reshape(DEG, NB, H4)
    whh = whh_ref[...]

    z = jnp.zeros((NB, D), jnp.float32)
    hs, c = z, z
    for t in range(DEG):
        gt = pin[t] + jnp.dot(hs, whh, precision=_PREC)
        i_ = _sigmoid(gt[:, 0:D])
        f_ = _sigmoid(gt[:, D:2 * D])
        g_ = jnp.tanh(gt[:, 2 * D:3 * D])
        o_ = _sigmoid(gt[:, 3 * D:4 * D])
        c = f_ * c + i_ * g_
        hs = o_ * jnp.tanh(c)

    out = (jnp.dot(h_blk, wself_ref[...], preferred_element_type=jnp.float32)
           + jnp.dot(hs.astype(jnp.bfloat16), wneigh_ref[...],
                     preferred_element_type=jnp.float32))
    if want_ls:
        m = jnp.max(out, axis=-1, keepdims=True)
        e = jnp.exp(out - m)
        out = out - m - jnp.log(jnp.sum(e, axis=-1, keepdims=True))
        out_ref[...] = out
    else:
        out_ref[...] = out
    if want_stats:
        stats_ref = maybe_stats[0]
        s1 = jnp.sum(out, axis=0, keepdims=True)
        s2 = jnp.sum(out * out, axis=0, keepdims=True)
        stats_ref[...] = jnp.concatenate([s1, s2], axis=0)[None]


def _tc_layer(h, g3, wih_t, whh_t, wself_t, wneigh_t, bias, scale, shift,
              *, cn, cnp, koff, apply_norm, want_stats, want_ls):
    nblk = cn // NB
    body = functools.partial(_layer_body, apply_norm=apply_norm,
                             want_stats=want_stats, want_ls=want_ls)
    in_specs = [
        pl.BlockSpec((NB, D), lambda i: (i + koff, 0)),
        pl.BlockSpec((DEG, NB, D), lambda i: (0, i, 0)),
        pl.BlockSpec((D, H4), lambda i: (0, 0)),
        pl.BlockSpec((D, H4), lambda i: (0, 0)),
        pl.BlockSpec((D, D), lambda i: (0, 0)),
        pl.BlockSpec((D, D), lambda i: (0, 0)),
        pl.BlockSpec((1, H4), lambda i: (0, 0)),
        pl.BlockSpec((1, D), lambda i: (0, 0)),
        pl.BlockSpec((1, D), lambda i: (0, 0)),
    ]
    if want_stats:
        out_shape = [jax.ShapeDtypeStruct((cn, D), jnp.float32),
                     jax.ShapeDtypeStruct((nblk, 2, D), jnp.float32)]
        out_specs = [pl.BlockSpec((NB, D), lambda i: (i, 0)),
                     pl.BlockSpec((1, 2, D), lambda i: (i, 0, 0))]
    else:
        out_shape = jax.ShapeDtypeStruct((cn, D), jnp.float32)
        out_specs = pl.BlockSpec((NB, D), lambda i: (i, 0))
    return pl.pallas_call(
        body,
        grid=(nblk,),
        in_specs=in_specs,
        out_specs=out_specs,
        out_shape=out_shape,
    )(h, g3, wih_t, whh_t, wself_t, wneigh_t, bias, scale, shift)


NC = 5


def kernel(x, edge_index, Wself, Wneigh, Wih, Whh, bih, bhh, gamma, beta):
    src = edge_index[0]
    cn = N // NC
    cnp = -(-cn // 256) * 256
    src_mat = src.reshape(N, DEG).T
    src_chunks = []
    for k in range(NC):
        part = src_mat[:, k * cn:(k + 1) * cn]
        part = jnp.pad(part, ((0, 0), (0, cnp - cn)))
        src_chunks.append(part.reshape(DEG * cnp // GW, 1, GW))
    wih_t = jnp.swapaxes(Wih, 1, 2).astype(jnp.bfloat16)
    whh_t = jnp.swapaxes(Whh, 1, 2)
    wself_t = jnp.swapaxes(Wself, 1, 2).astype(jnp.bfloat16)
    wneigh_t = jnp.swapaxes(Wneigh, 1, 2).astype(jnp.bfloat16)
    bias = (bih + bhh).reshape(L, 1, H4)

    h = x
    scale = jnp.ones((1, D), jnp.float32)
    shift = jnp.zeros((1, D), jnp.float32)
    for l in range(L):
        last = l == L - 1
        outs, stats_parts = [], []
        for k in range(NC):
            g3 = _sc_gather(h, src_chunks[k]).reshape(DEG, cnp, D)
            res = _tc_layer(h, g3, wih_t[l], whh_t[l], wself_t[l],
                            wneigh_t[l], bias[l], scale, shift,
                            cn=cn, cnp=cnp, koff=k * (cn // NB),
                            apply_norm=(l > 0), want_stats=not last,
                            want_ls=last)
            if last:
                outs.append(res)
            else:
                outs.append(res[0])
                stats_parts.append(res[1])
        h = jnp.concatenate(outs, axis=0)
        if last:
            return h
        s = jnp.sum(jnp.concatenate(stats_parts, axis=0), axis=0)
        mu = s[0] / N
        var = s[1] / N - mu * mu
        inv = lax.rsqrt(var + 1e-5)
        scale = (gamma[l] * inv).reshape(1, D)
        shift = (beta[l] - mu * gamma[l] * inv).reshape(1, D)
    return None
